# baseline (speedup 1.0000x reference)
import functools

import jax
import jax.numpy as jnp
from jax import lax
from jax.experimental import pallas as pl
from jax.experimental.pallas import tpu as pltpu
from jax.experimental.pallas import tpu_sc as plsc

_NUM_WORKERS = 32
_CHUNK = 128


def _sc_gather_concat(fans_table, avgrating_table, fidx, aidx):
    b = fidx.shape[0] * fidx.shape[1] * fidx.shape[2]
    d = fans_table.shape[1]
    f_out = 2 * d
    bpw = b // _NUM_WORKERS
    nch = bpw // _CHUNK
    n_tr = f_out // 8
    n_tc = b // _CHUNK
    mesh = plsc.VectorSubcoreMesh(core_axis_name="c", subcore_axis_name="s")

    @functools.partial(
        pl.kernel,
        mesh=mesh,
        compiler_params=pltpu.CompilerParams(
            use_tc_tiling_on_sc=False, needs_layout_passes=False),
        out_type=jax.ShapeDtypeStruct((b * f_out,), jnp.float32),
        scratch_types=[
            pltpu.VMEM((nch, _CHUNK), jnp.int32),
            pltpu.VMEM((nch, _CHUNK), jnp.int32),
            pltpu.VMEM((bpw, d), jnp.float32),
            pltpu.VMEM((bpw, d), jnp.float32),
            pltpu.VMEM((n_tr * nch * 8 * _CHUNK,), jnp.float32),
            pltpu.SemaphoreType.DMA,
        ],
    )
    def k(fans_hbm, avg_hbm, fidx_hbm, aidx_hbm, out_hbm,
          fidx_v, aidx_v, frows, arows, tp, sem):
        wid = lax.axis_index("s") * 2 + lax.axis_index("c")
        pltpu.sync_copy(fidx_hbm.at[wid], fidx_v)
        pltpu.sync_copy(aidx_hbm.at[wid], aidx_v)
        copies = []
        for t in range(nch):
            sl = pl.ds(t * _CHUNK, _CHUNK)
            copies.append(
                pltpu.async_copy(fans_hbm.at[fidx_v.at[t]], frows.at[sl], sem))
            copies.append(
                pltpu.async_copy(avg_hbm.at[aidx_v.at[t]], arows.at[sl], sem))
        for c in copies:
            c.wait()

        iota = lax.iota(jnp.int32, 16)
        seg = nch * 8 * _CHUNK
        tr_base = (iota >> 3) * seg + (iota & 7) * _CHUNK
        offs = [tr_base + 2 * half * seg for half in range(4)]

        @pl.loop(0, nch)
        def _k_loop(kk):
            @pl.loop(0, _CHUNK)
            def _l_loop(l):
                r = kk * _CHUNK + l
                soff = jnp.full((16,), kk * (8 * _CHUNK) + l, jnp.int32)
                plsc.store_scatter(tp, [offs[0] + soff], frows[r, pl.ds(0, 16)])
                plsc.store_scatter(tp, [offs[1] + soff], frows[r, pl.ds(16, 16)])
                plsc.store_scatter(tp, [offs[2] + soff], arows[r, pl.ds(0, 16)])
                plsc.store_scatter(tp, [offs[3] + soff], arows[r, pl.ds(16, 16)])

        seg = nch * 8 * _CHUNK
        out_copies = []
        for tr in range(n_tr):
            out_copies.append(pltpu.async_copy(
                tp.at[pl.ds(tr * seg, seg)],
                out_hbm.at[pl.ds(tr * (8 * b) + wid * seg, seg)],
                sem))
        for c in out_copies:
            c.wait()

    return k(fans_table, avgrating_table, fidx, aidx)


def kernel(user_fea, fans_table, avgrating_table):
    b = user_fea.shape[0]
    d = fans_table.shape[1]
    bpw = b // _NUM_WORKERS
    nch = bpw // _CHUNK
    fidx = user_fea[:, 1].astype(jnp.int32).reshape(_NUM_WORKERS, nch, _CHUNK)
    aidx = user_fea[:, 2].astype(jnp.int32).reshape(_NUM_WORKERS, nch, _CHUNK)
    y = _sc_gather_concat(fans_table, avgrating_table, fidx, aidx)
    y = y.reshape(2 * d // 8, b // _CHUNK, 8, _CHUNK)
    return y.transpose(1, 3, 0, 2).reshape(b, 2 * d)
